# baseline (speedup 1.0000x reference)
import functools

import jax
import jax.numpy as jnp
from jax import lax
from jax.experimental import pallas as pl
from jax.experimental.pallas import tpu as pltpu
from jax.experimental.pallas import tpu_sc as plsc

NUM_USERS = 1000000
NUM_ITEMS = 1000000
EMB = 64
BATCH = 16384

NC = 2
NS = 16
NW = NC * NS
B_PER_W = BATCH // NW
CHUNK = 128
NCHUNK = B_PER_W // CHUNK


def _gather_body(users_hbm, items_hbm, ut_hbm, it_hbm, ue_hbm, ie_hbm,
                 uidx_v, iidx_v, urows_v, irows_v, sem):
    wid = lax.axis_index("s") * NC + lax.axis_index("c")
    pltpu.sync_copy(users_hbm.at[wid], uidx_v)
    pltpu.sync_copy(items_hbm.at[wid], iidx_v)
    copies = []
    for j in range(NCHUNK):
        copies.append(pltpu.async_copy(
            ut_hbm.at[uidx_v.at[j]], urows_v.at[pl.ds(j * CHUNK, CHUNK)], sem))
        copies.append(pltpu.async_copy(
            it_hbm.at[iidx_v.at[j]], irows_v.at[pl.ds(j * CHUNK, CHUNK)], sem))
    for c in copies:
        c.wait()
    pltpu.sync_copy(urows_v, ue_hbm.at[wid])
    pltpu.sync_copy(irows_v, ie_hbm.at[wid])


@functools.cache
def _sc_gather():
    return pl.kernel(
        _gather_body,
        mesh=plsc.VectorSubcoreMesh(core_axis_name="c", subcore_axis_name="s"),
        out_type=[
            jax.ShapeDtypeStruct((NW, B_PER_W, EMB), jnp.float32),
            jax.ShapeDtypeStruct((NW, B_PER_W, EMB), jnp.float32),
        ],
        scratch_types=[
            pltpu.VMEM((NCHUNK, CHUNK), jnp.int32),
            pltpu.VMEM((NCHUNK, CHUNK), jnp.int32),
            pltpu.VMEM((B_PER_W, EMB), jnp.float32),
            pltpu.VMEM((B_PER_W, EMB), jnp.float32),
            pltpu.SemaphoreType.DMA,
        ],
        compiler_params=pltpu.CompilerParams(use_tc_tiling_on_sc=False),
    )


BLK = 1024
GRID = BATCH // BLK


def _mlp_body(ue_ref, ie_ref, w0u_ref, w0i_ref, b0_ref, w1_ref, b1_ref,
              w2_ref, b2_ref, w3_ref, b3_ref, out_ref):
    h = jnp.dot(ue_ref[...], w0u_ref[...], preferred_element_type=jnp.float32)
    h = h + jnp.dot(ie_ref[...], w0i_ref[...], preferred_element_type=jnp.float32)
    h = jnp.maximum(h + b0_ref[...], 0.0)
    h = jnp.maximum(
        jnp.dot(h, w1_ref[...], preferred_element_type=jnp.float32) + b1_ref[...], 0.0)
    h = jnp.maximum(
        jnp.dot(h, w2_ref[...], preferred_element_type=jnp.float32) + b2_ref[...], 0.0)
    out_ref[...] = jnp.sum(h * w3_ref[...], axis=1) + b3_ref[...]


def _mlp(ue, ie, w0u, w0i, b0, w1t, b1, w2t, b2, w3, b3):
    full = lambda shape: pl.BlockSpec(shape, lambda i: (0,) * len(shape))
    return pl.pallas_call(
        _mlp_body,
        grid=(GRID,),
        in_specs=[
            pl.BlockSpec((BLK, EMB), lambda i: (i, 0)),
            pl.BlockSpec((BLK, EMB), lambda i: (i, 0)),
            full((EMB, 128)),
            full((EMB, 128)),
            full((128,)),
            full((128, 64)),
            full((64,)),
            full((64, 32)),
            full((32,)),
            full((32,)),
            full((1,)),
        ],
        out_specs=pl.BlockSpec((BLK,), lambda i: (i,)),
        out_shape=jax.ShapeDtypeStruct((BATCH,), jnp.float32),
        compiler_params=pltpu.CompilerParams(
            dimension_semantics=("parallel",)),
    )(ue, ie, w0u, w0i, b0, w1t, b1, w2t, b2, w3, b3)


def kernel(users, items, user_table, item_table, W0, b0, W1, b1, W2, b2, W3, b3):
    users_g = jnp.clip(users, 0, NUM_USERS - 1).astype(jnp.int32)
    items_g = jnp.clip(items, 0, NUM_ITEMS - 1).astype(jnp.int32)
    users_g = users_g.reshape(NW, NCHUNK, CHUNK)
    items_g = items_g.reshape(NW, NCHUNK, CHUNK)
    ue, ie = _sc_gather()(users_g, items_g, user_table, item_table)
    ue = ue.reshape(BATCH, EMB)
    ie = ie.reshape(BATCH, EMB)
    out = _mlp(ue, ie,
               W0[:, :EMB].T, W0[:, EMB:].T, b0,
               W1.T, b1, W2.T, b2, W3[0], b3)
    return out
